# Initial kernel scaffold; baseline (speedup 1.0000x reference)
import jax
import jax.numpy as jnp
from jax.experimental import pallas as pl


def kernel(x, edge_index, edge_attr, W_node, b_node, W_edge, b_edge, W_att, b_att):
    raise NotImplementedError("write your pallas kernel here")


# speedup vs baseline: 16.9883x; 16.9883x over previous
import jax
import jax.numpy as jnp
from jax import lax
from jax.experimental import pallas as pl
from jax.experimental.pallas import tpu as pltpu
from jax.experimental.pallas import tpu_sc as plsc

N = 10000
E = 160000
D = 128
K = 64

NC = 2
NS = 16
L = 16
NW = NC * NS
EPAD = 163840
PER_W = EPAD // NW
CH = PER_W // 128
NPAD = EPAD - E
MAP_SIZE = N * N + NPAD
FIRE = 8

_mesh = plsc.VectorSubcoreMesh(core_axis_name="c", subcore_axis_name="s",
                               num_cores=NC, num_subcores=NS)


def _worker_id():
    return lax.axis_index("s") * NC + lax.axis_index("c")


def _stage_keys_ids(row_hbm, col_hbm, row_v, col_v, keys_v, ids_v):
    wid = _worker_id()
    r0 = wid * CH
    pltpu.sync_copy(row_hbm.at[pl.ds(r0, CH)], row_v)
    pltpu.sync_copy(col_hbm.at[pl.ds(r0, CH)], col_v)
    base = wid * PER_W
    lane = lax.iota(jnp.int32, L)

    def body(i, carry):
        j = i // 8
        o = (i % 8) * L
        r = row_v[j, pl.ds(o, L)]
        c = col_v[j, pl.ds(o, L)]
        keys_v[j, pl.ds(o, L)] = r * N + c
        ids_v[j, pl.ds(o, L)] = base + i * L + lane
        return carry

    lax.fori_loop(0, CH * 8, body, 0)


def _scatter_body(row_hbm, col_hbm, map_hbm, row_v, col_v, keys_v, ids_v, sem):
    _stage_keys_ids(row_hbm, col_hbm, row_v, col_v, keys_v, ids_v)
    handles = []
    for j in range(CH):
        handles.append(
            pltpu.async_copy(ids_v.at[j], map_hbm.at[keys_v.at[j]], sem))
        if len(handles) == FIRE:
            for h in handles:
                h.wait()
            handles = []
    for h in handles:
        h.wait()


def _count_body(row_hbm, col_hbm, map_hbm, cnt_hbm,
                row_v, col_v, keys_v, ids_v, got_v, acc_v, sem):
    _stage_keys_ids(row_hbm, col_hbm, row_v, col_v, keys_v, ids_v)
    handles = []
    for j in range(CH):
        handles.append(
            pltpu.async_copy(map_hbm.at[keys_v.at[j]], got_v.at[j], sem))
        if len(handles) == FIRE:
            for h in handles:
                h.wait()
            handles = []
    for h in handles:
        h.wait()

    def body(i, acc):
        j = i // 8
        o = (i % 8) * L
        g = got_v[j, pl.ds(o, L)]
        d = ids_v[j, pl.ds(o, L)]
        one = jnp.ones((L,), jnp.float32)
        zero = jnp.zeros((L,), jnp.float32)
        return acc + jnp.where(g == d, one, zero)

    acc = lax.fori_loop(0, CH * 8, body, jnp.zeros((L,), jnp.float32))
    acc_v[pl.ds(0, L)] = acc
    zero = jnp.zeros((L,), jnp.float32)
    for t in range(1, 128 // L):
        acc_v[pl.ds(t * L, L)] = zero
    wid = _worker_id()
    pltpu.sync_copy(acc_v, cnt_hbm.at[wid])


def _colsum_body(x_ref, o_ref):
    s = jnp.sum(x_ref[...], axis=0, keepdims=True)
    o_ref[...] = jnp.broadcast_to(s * (1.0 / K), (K, D))


def _ew_body(cnt_ref, o_ref):
    total = jnp.sum(cnt_ref[...])
    w = (total - float(NPAD)) * (1.0 / (K * K))
    o_ref[...] = jnp.full((NW, 128), w, jnp.float32)


def kernel(x, edge_index, edge_attr, W_node, b_node, W_edge, b_edge,
           W_att, b_att):
    row = edge_index[0]
    col = edge_index[1]
    pad_r = jnp.full((NPAD,), N, jnp.int32)
    pad_c = lax.iota(jnp.int32, NPAD)
    rowp = jnp.concatenate([row, pad_r]).reshape(EPAD // 128, 128)
    colp = jnp.concatenate([col, pad_c]).reshape(EPAD // 128, 128)

    scatter_k = pl.kernel(
        _scatter_body,
        out_type=jax.ShapeDtypeStruct((MAP_SIZE,), jnp.int32),
        mesh=_mesh,
        scratch_types=[pltpu.VMEM((CH, 128), jnp.int32)] * 4
        + [pltpu.SemaphoreType.DMA],
    )
    map_arr = scatter_k(rowp, colp)

    count_k = pl.kernel(
        _count_body,
        out_type=jax.ShapeDtypeStruct((NW, 128), jnp.float32),
        mesh=_mesh,
        scratch_types=[pltpu.VMEM((CH, 128), jnp.int32)] * 5
        + [pltpu.VMEM((128,), jnp.float32), pltpu.SemaphoreType.DMA],
    )
    counts = count_k(rowp, colp, map_arr)

    x_pooled = pl.pallas_call(
        _colsum_body,
        out_shape=jax.ShapeDtypeStruct((K, D), jnp.float32),
    )(x)
    ew2d = pl.pallas_call(
        _ew_body,
        out_shape=jax.ShapeDtypeStruct((NW, 128), jnp.float32),
    )(counts)

    grid = lax.iota(jnp.int32, K * K)
    edge_index_pooled = jnp.stack([grid // K, grid % K], axis=0)
    edge_weight_pooled = ew2d.reshape(K * K)
    batch_pooled = jnp.zeros((K,), jnp.int32)
    return (x_pooled, edge_index_pooled, edge_weight_pooled, batch_pooled)


# speedup vs baseline: 17.2257x; 1.0140x over previous
import jax
import jax.numpy as jnp
from jax import lax
from jax.experimental import pallas as pl
from jax.experimental.pallas import tpu as pltpu
from jax.experimental.pallas import tpu_sc as plsc

N = 10000
E = 160000
D = 128
K = 64

NC = 2
NS = 16
L = 16
NW = NC * NS
EPAD = 163840
PER_W = EPAD // NW
CH = PER_W // 128
NPAD = EPAD - E
MAP_SIZE = N * N + NPAD
FIRE = 40

_mesh = plsc.VectorSubcoreMesh(core_axis_name="c", subcore_axis_name="s",
                               num_cores=NC, num_subcores=NS)


def _worker_id():
    return lax.axis_index("s") * NC + lax.axis_index("c")


def _stage_keys_ids(row_hbm, col_hbm, row_v, col_v, keys_v, ids_v):
    wid = _worker_id()
    r0 = wid * CH
    pltpu.sync_copy(row_hbm.at[pl.ds(r0, CH)], row_v)
    pltpu.sync_copy(col_hbm.at[pl.ds(r0, CH)], col_v)
    base = wid * PER_W
    lane = lax.iota(jnp.int32, L)

    def body(i, carry):
        j = i // 8
        o = (i % 8) * L
        r = row_v[j, pl.ds(o, L)]
        c = col_v[j, pl.ds(o, L)]
        keys_v[j, pl.ds(o, L)] = r * N + c
        ids_v[j, pl.ds(o, L)] = base + i * L + lane
        return carry

    lax.fori_loop(0, CH * 8, body, 0)


def _scatter_body(row_hbm, col_hbm, map_hbm, row_v, col_v, keys_v, ids_v, sem):
    _stage_keys_ids(row_hbm, col_hbm, row_v, col_v, keys_v, ids_v)
    handles = []
    for j in range(CH):
        handles.append(
            pltpu.async_copy(ids_v.at[j], map_hbm.at[keys_v.at[j]], sem))
        if len(handles) == FIRE:
            for h in handles:
                h.wait()
            handles = []
    for h in handles:
        h.wait()


def _count_body(row_hbm, col_hbm, map_hbm, cnt_hbm,
                row_v, col_v, keys_v, ids_v, got_v, acc_v, sem):
    _stage_keys_ids(row_hbm, col_hbm, row_v, col_v, keys_v, ids_v)
    handles = []
    for j in range(CH):
        handles.append(
            pltpu.async_copy(map_hbm.at[keys_v.at[j]], got_v.at[j], sem))
        if len(handles) == FIRE:
            for h in handles:
                h.wait()
            handles = []
    for h in handles:
        h.wait()

    def body(i, acc):
        j = i // 8
        o = (i % 8) * L
        g = got_v[j, pl.ds(o, L)]
        d = ids_v[j, pl.ds(o, L)]
        one = jnp.ones((L,), jnp.float32)
        zero = jnp.zeros((L,), jnp.float32)
        return acc + jnp.where(g == d, one, zero)

    acc = lax.fori_loop(0, CH * 8, body, jnp.zeros((L,), jnp.float32))
    acc_v[pl.ds(0, L)] = acc
    zero = jnp.zeros((L,), jnp.float32)
    for t in range(1, 128 // L):
        acc_v[pl.ds(t * L, L)] = zero
    wid = _worker_id()
    pltpu.sync_copy(acc_v, cnt_hbm.at[wid])


def _colsum_body(x_ref, o_ref):
    s = jnp.sum(x_ref[...], axis=0, keepdims=True)
    o_ref[...] = jnp.broadcast_to(s * (1.0 / K), (K, D))


def _ew_body(cnt_ref, o_ref):
    total = jnp.sum(cnt_ref[...])
    w = (total - float(NPAD)) * (1.0 / (K * K))
    o_ref[...] = jnp.full((NW, 128), w, jnp.float32)


def kernel(x, edge_index, edge_attr, W_node, b_node, W_edge, b_edge,
           W_att, b_att):
    row = edge_index[0]
    col = edge_index[1]
    pad_r = jnp.full((NPAD,), N, jnp.int32)
    pad_c = lax.iota(jnp.int32, NPAD)
    rowp = jnp.concatenate([row, pad_r]).reshape(EPAD // 128, 128)
    colp = jnp.concatenate([col, pad_c]).reshape(EPAD // 128, 128)

    scatter_k = pl.kernel(
        _scatter_body,
        out_type=jax.ShapeDtypeStruct((MAP_SIZE,), jnp.int32),
        mesh=_mesh,
        scratch_types=[pltpu.VMEM((CH, 128), jnp.int32)] * 4
        + [pltpu.SemaphoreType.DMA],
    )
    map_arr = scatter_k(rowp, colp)

    count_k = pl.kernel(
        _count_body,
        out_type=jax.ShapeDtypeStruct((NW, 128), jnp.float32),
        mesh=_mesh,
        scratch_types=[pltpu.VMEM((CH, 128), jnp.int32)] * 5
        + [pltpu.VMEM((128,), jnp.float32), pltpu.SemaphoreType.DMA],
    )
    counts = count_k(rowp, colp, map_arr)

    x_pooled = pl.pallas_call(
        _colsum_body,
        out_shape=jax.ShapeDtypeStruct((K, D), jnp.float32),
    )(x)
    ew2d = pl.pallas_call(
        _ew_body,
        out_shape=jax.ShapeDtypeStruct((NW, 128), jnp.float32),
    )(counts)

    grid = lax.iota(jnp.int32, K * K)
    edge_index_pooled = jnp.stack([grid // K, grid % K], axis=0)
    edge_weight_pooled = ew2d.reshape(K * K)
    batch_pooled = jnp.zeros((K,), jnp.int32)
    return (x_pooled, edge_index_pooled, edge_weight_pooled, batch_pooled)


# speedup vs baseline: 35.9321x; 2.0860x over previous
import jax
import jax.numpy as jnp
from jax import lax
from jax.experimental import pallas as pl
from jax.experimental.pallas import tpu as pltpu
from jax.experimental.pallas import tpu_sc as plsc

N = 10000
E = 160000
D = 128
K = 64

NC = 2
NS = 16
L = 16
NW = NC * NS
EPAD = 163840
PER_W = EPAD // NW
CH = PER_W // 128
SCH = 2 * CH
NPAD = EPAD - E
PADK0 = N * N + NPAD
MAP_SIZE = PADK0 + EPAD
FIRE = 8

ZCH = 8192
HBLK = 160
HSIZE = HBLK * ZCH
HFOLD = (1 << 21) - HSIZE
HMUL1 = -1640531527
HMUL2 = -862048943

_mesh = plsc.VectorSubcoreMesh(core_axis_name="c", subcore_axis_name="s",
                               num_cores=NC, num_subcores=NS)


def _hash16(k, mul, shift):
    h = lax.shift_right_logical(k * mul, shift)
    h = h & ((1 << 21) - 1)
    return jnp.where(h >= HSIZE, h - HFOLD, h)


def _prefix16(v, lane):
    cum = v
    for dsh in (1, 2, 4, 8):
        idx = jnp.maximum(lane - dsh, 0)
        sh = cum.at[idx].get(mode="promise_in_bounds")
        cum = cum + jnp.where(lane >= dsh, sh, 0)
    return cum


def _classify_body(rowp_hbm, colp_hbm, map_hbm, skeys_hbm, sids_hbm,
                   svalid_hbm, uniq_hbm,
                   rowb, colb, hashb, hashb2,
                   skey1, sid1, sval1, zbuf, onesb, urow, hist, sem):
    cid = lax.axis_index("c")
    sid_ = lax.axis_index("s")
    wid = sid_ * NC + cid
    lane = lax.iota(jnp.int32, L)
    zero16 = jnp.zeros((L,), jnp.int32)

    def z_body(i, carry):
        zbuf[pl.ds(i * L, L)] = zero16
        return carry

    lax.fori_loop(0, ZCH // L, z_body, 0)
    for bi in range(HBLK // NS):
        b = sid_ * (HBLK // NS) + bi
        pltpu.sync_copy(zbuf, hist.at[pl.ds(b * ZCH, ZCH)])

    def ones_body(i, carry):
        onesb[pl.ds(i * L, L)] = jnp.ones((L,), jnp.int32)
        return carry

    lax.fori_loop(0, 128 // L, ones_body, 0)

    def stage_and_hash(first_row):
        pltpu.sync_copy(rowp_hbm.at[pl.ds(first_row, CH)], rowb)
        pltpu.sync_copy(colp_hbm.at[pl.ds(first_row, CH)], colb)

        def keys_body(i, carry):
            j = i // 8
            o = (i % 8) * L
            k = rowb[j, pl.ds(o, L)] * N + colb[j, pl.ds(o, L)]
            hashb[j, pl.ds(o, L)] = _hash16(k, HMUL1, 10)
            hashb2[j, pl.ds(o, L)] = _hash16(k, HMUL2, 9)
            return carry

        lax.fori_loop(0, CH * 8, keys_body, 0)

    plsc.subcore_barrier()

    for phase in range(2):
        stage_and_hash(sid_ * SCH + phase * CH)
        handles = []
        for hb in (hashb, hashb2):
            for j in range(CH):
                handles.append(
                    pltpu.async_copy(onesb, hist.at[hb.at[j]], sem,
                                     add=True))
                if len(handles) == FIRE:
                    for h in handles:
                        h.wait()
                    handles = []
        for h in handles:
            h.wait()

    plsc.subcore_barrier()

    stage_and_hash(sid_ * SCH + cid * CH)
    handles = []
    for hb, cb in ((hashb, zbuf), (hashb2, sval1)):
        for j in range(CH):
            handles.append(
                pltpu.async_copy(hist.at[hb.at[j]],
                                 cb.at[pl.ds(j * 128, 128)], sem))
            if len(handles) == FIRE:
                for h in handles:
                    h.wait()
                handles = []
    for h in handles:
        h.wait()

    padbase = PADK0 + wid * PER_W

    def pre_body(i, carry):
        skey1[pl.ds(i * L, L)] = padbase + i * L + lane
        sid1[pl.ds(i * L, L)] = jnp.full((L,), -1, jnp.int32)
        return carry

    lax.fori_loop(0, PER_W // L, pre_body, 0)

    idbase = wid * PER_W

    def cls_body(i, off):
        j = i // 8
        o = (i % 8) * L
        k16 = rowb[j, pl.ds(o, L)] * N + colb[j, pl.ds(o, L)]
        c1 = zbuf[pl.ds(i * L, L)]
        c2 = sval1[pl.ds(i * L, L)]
        d16 = idbase + i * L + lane
        mi = jnp.where((c1 >= 2) & (c2 >= 2), 1, 0)
        total0 = _prefix16(mi, lane)[15]

        def wbody(it, carry):
            off_, mv = carry
            cum = _prefix16(mv, lane)
            sel = jnp.where((mv == 1) & (cum == 1), lane, 0)
            f = _prefix16(sel, lane)[15]
            fs = jnp.full((L,), f, jnp.int32)
            kv = k16.at[fs].get(mode="promise_in_bounds")
            dv = d16.at[fs].get(mode="promise_in_bounds")
            skey1[pl.ds(off_, L)] = kv
            sid1[pl.ds(off_, L)] = dv
            return (off_ + 1, jnp.where(lane == f, 0, mv))

        off2, _ = lax.fori_loop(0, total0, wbody, (off, mi))
        return off2

    ns = lax.fori_loop(0, CH * 8, cls_body, jnp.int32(0))

    def val_body(i, carry):
        v = jnp.where(i * L + lane < ns, 1, 0).astype(jnp.int32)
        sval1[pl.ds(i * L, L)] = v
        return carry

    lax.fori_loop(0, PER_W // L, val_body, 0)

    ng = (ns + L - 1) // L

    def sc_body(g, carry):
        idx16 = skey1[pl.ds(g * L, L)]
        pltpu.async_copy(sid1.at[pl.ds(g * L, L)],
                         map_hbm.at[idx16], sem).wait()
        return carry

    lax.fori_loop(0, ng, sc_body, 0)

    uq = jnp.full((L,), PER_W, jnp.int32) - ns
    urow[pl.ds(0, L)] = jnp.where(lane == 0, uq, 0)
    for t in range(1, 128 // L):
        urow[pl.ds(t * L, L)] = zero16
    pltpu.sync_copy(urow, uniq_hbm.at[wid])
    pltpu.sync_copy(skey1.at[pl.ds(0, PER_W)],
                    skeys_hbm.at[pl.ds(wid * PER_W, PER_W)])
    pltpu.sync_copy(sid1.at[pl.ds(0, PER_W)],
                    sids_hbm.at[pl.ds(wid * PER_W, PER_W)])
    pltpu.sync_copy(sval1, svalid_hbm.at[pl.ds(wid * PER_W, PER_W)])


def _count_body(skeys_hbm, sids_hbm, svalid_hbm, map_hbm, cnt_hbm,
                skb, sdb, svb, gotb, acc_v, sem):
    cid = lax.axis_index("c")
    sid_ = lax.axis_index("s")
    wid = sid_ * NC + cid
    base = wid * PER_W
    pltpu.sync_copy(skeys_hbm.at[pl.ds(base, PER_W)], skb)
    pltpu.sync_copy(sids_hbm.at[pl.ds(base, PER_W)], sdb)
    pltpu.sync_copy(svalid_hbm.at[pl.ds(base, PER_W)], svb)

    handles = []
    for j in range(CH):
        handles.append(
            pltpu.async_copy(map_hbm.at[skb.at[pl.ds(j * 128, 128)]],
                             gotb.at[pl.ds(j * 128, 128)], sem))
        if len(handles) == FIRE:
            for h in handles:
                h.wait()
            handles = []
    for h in handles:
        h.wait()

    def body(i, acc):
        o = i * L
        g = gotb[pl.ds(o, L)]
        d = sdb[pl.ds(o, L)]
        v = svb[pl.ds(o, L)]
        one = jnp.ones((L,), jnp.float32)
        zero = jnp.zeros((L,), jnp.float32)
        return acc + jnp.where((g == d) & (v == 1), one, zero)

    acc = lax.fori_loop(0, PER_W // L, body, jnp.zeros((L,), jnp.float32))
    acc_v[pl.ds(0, L)] = acc
    zero = jnp.zeros((L,), jnp.float32)
    for t in range(1, 128 // L):
        acc_v[pl.ds(t * L, L)] = zero
    pltpu.sync_copy(acc_v, cnt_hbm.at[wid])


def _colsum_body(x_ref, o_ref):
    s = jnp.sum(x_ref[...], axis=0, keepdims=True)
    o_ref[...] = jnp.broadcast_to(s * (1.0 / K), (K, D))


def _ew_body(cnt_ref, uniq_ref, o_ref):
    total = jnp.sum(cnt_ref[...]) + jnp.sum(uniq_ref[...].astype(jnp.float32))
    w = (total - float(NPAD)) * (1.0 / (K * K))
    o_ref[...] = jnp.full((NW, 128), w, jnp.float32)


def kernel(x, edge_index, edge_attr, W_node, b_node, W_edge, b_edge,
           W_att, b_att):
    row = edge_index[0]
    col = edge_index[1]
    pad_r = jnp.full((NPAD,), N, jnp.int32)
    pad_c = lax.iota(jnp.int32, NPAD)
    rowp = jnp.concatenate([row, pad_r]).reshape(EPAD // 128, 128)
    colp = jnp.concatenate([col, pad_c]).reshape(EPAD // 128, 128)

    classify_k = pl.kernel(
        _classify_body,
        out_type=(
            jax.ShapeDtypeStruct((MAP_SIZE,), jnp.int32),
            jax.ShapeDtypeStruct((EPAD,), jnp.int32),
            jax.ShapeDtypeStruct((EPAD,), jnp.int32),
            jax.ShapeDtypeStruct((EPAD,), jnp.int32),
            jax.ShapeDtypeStruct((NW, 128), jnp.int32),
        ),
        mesh=_mesh,
        scratch_types=[
            pltpu.VMEM((CH, 128), jnp.int32),
            pltpu.VMEM((CH, 128), jnp.int32),
            pltpu.VMEM((CH, 128), jnp.int32),
            pltpu.VMEM((CH, 128), jnp.int32),
            pltpu.VMEM((PER_W + L,), jnp.int32),
            pltpu.VMEM((PER_W + L,), jnp.int32),
            pltpu.VMEM((PER_W,), jnp.int32),
            pltpu.VMEM((ZCH,), jnp.int32),
            pltpu.VMEM((128,), jnp.int32),
            pltpu.VMEM((128,), jnp.int32),
            pltpu.VMEM_SHARED((HSIZE,), jnp.int32),
            pltpu.SemaphoreType.DMA,
        ],
    )
    map_arr, skeys, sids, svalid, uniq = classify_k(rowp, colp)

    count_k = pl.kernel(
        _count_body,
        out_type=jax.ShapeDtypeStruct((NW, 128), jnp.float32),
        mesh=_mesh,
        scratch_types=[pltpu.VMEM((PER_W,), jnp.int32)] * 4
        + [pltpu.VMEM((128,), jnp.float32), pltpu.SemaphoreType.DMA],
    )
    counts = count_k(skeys, sids, svalid, map_arr)

    x_pooled = pl.pallas_call(
        _colsum_body,
        out_shape=jax.ShapeDtypeStruct((K, D), jnp.float32),
    )(x)
    ew2d = pl.pallas_call(
        _ew_body,
        out_shape=jax.ShapeDtypeStruct((NW, 128), jnp.float32),
    )(counts, uniq)

    grid = lax.iota(jnp.int32, K * K)
    edge_index_pooled = jnp.stack([grid // K, grid % K], axis=0)
    edge_weight_pooled = ew2d.reshape(K * K)
    batch_pooled = jnp.zeros((K,), jnp.int32)
    return (x_pooled, edge_index_pooled, edge_weight_pooled, batch_pooled)


# speedup vs baseline: 40.4601x; 1.1260x over previous
import jax
import jax.numpy as jnp
from jax import lax
from jax.experimental import pallas as pl
from jax.experimental.pallas import tpu as pltpu
from jax.experimental.pallas import tpu_sc as plsc

N = 10000
E = 160000
D = 128
K = 64

NC = 2
NS = 16
L = 16
NW = NC * NS
EPAD = 163840
PER_W = EPAD // NW
CH = PER_W // 128
SCH = 2 * CH
NPAD = EPAD - E
PADK0 = N * N + NPAD
MAP_SIZE = PADK0 + EPAD
FIRE = 8

ZCH = 8192
HBLK = 160
HSIZE = HBLK * ZCH
HFOLD = (1 << 21) - HSIZE
HMUL1 = -1640531527
HMUL2 = -862048943

_mesh = plsc.VectorSubcoreMesh(core_axis_name="c", subcore_axis_name="s",
                               num_cores=NC, num_subcores=NS)


def _hash16(k, mul, shift):
    h = lax.shift_right_logical(k * mul, shift)
    h = h & ((1 << 21) - 1)
    return jnp.where(h >= HSIZE, h - HFOLD, h)


def _prefix16(v, lane):
    cum = v
    for dsh in (1, 2, 4, 8):
        idx = jnp.maximum(lane - dsh, 0)
        sh = cum.at[idx].get(mode="promise_in_bounds")
        cum = cum + jnp.where(lane >= dsh, sh, 0)
    return cum


def _classify_body(rowp_hbm, colp_hbm, map_hbm, skeys_hbm, sids_hbm,
                   svalid_hbm, uniq_hbm,
                   rowb, colb, hashb, hashb2,
                   skey1, sid1, sval1, zbuf, onesb, urow, hist, sem):
    cid = lax.axis_index("c")
    sid_ = lax.axis_index("s")
    wid = sid_ * NC + cid
    lane = lax.iota(jnp.int32, L)
    zero16 = jnp.zeros((L,), jnp.int32)

    def z_body(i, carry):
        zbuf[pl.ds(i * L, L)] = zero16
        return carry

    lax.fori_loop(0, ZCH // L, z_body, 0)
    for bi in range(HBLK // NS):
        b = sid_ * (HBLK // NS) + bi
        pltpu.sync_copy(zbuf, hist.at[pl.ds(b * ZCH, ZCH)])

    def ones_body(i, carry):
        onesb[pl.ds(i * L, L)] = jnp.ones((L,), jnp.int32)
        return carry

    lax.fori_loop(0, 128 // L, ones_body, 0)

    def stage_and_hash(first_row):
        pltpu.sync_copy(rowp_hbm.at[pl.ds(first_row, CH)], rowb)
        pltpu.sync_copy(colp_hbm.at[pl.ds(first_row, CH)], colb)

        def keys_body(i, carry):
            j = i // 8
            o = (i % 8) * L
            k = rowb[j, pl.ds(o, L)] * N + colb[j, pl.ds(o, L)]
            hashb[j, pl.ds(o, L)] = _hash16(k, HMUL1, 10)
            hashb2[j, pl.ds(o, L)] = _hash16(k, HMUL2, 9)
            return carry

        lax.fori_loop(0, CH * 8, keys_body, 0)

    plsc.subcore_barrier()

    for phase in range(2):
        stage_and_hash(sid_ * SCH + phase * CH)
        handles = []
        for hb in (hashb, hashb2):
            for j in range(CH):
                handles.append(
                    pltpu.async_copy(onesb, hist.at[hb.at[j]], sem,
                                     add=True))
                if len(handles) == FIRE:
                    for h in handles:
                        h.wait()
                    handles = []
        for h in handles:
            h.wait()

    plsc.subcore_barrier()

    stage_and_hash(sid_ * SCH + cid * CH)
    handles = []
    for hb, cb in ((hashb, zbuf), (hashb2, sval1)):
        for j in range(CH):
            handles.append(
                pltpu.async_copy(hist.at[hb.at[j]],
                                 cb.at[pl.ds(j * 128, 128)], sem))
            if len(handles) == FIRE:
                for h in handles:
                    h.wait()
                handles = []
    for h in handles:
        h.wait()

    padbase = PADK0 + wid * PER_W

    def pre_body(i, carry):
        skey1[pl.ds(i * L, L)] = padbase + i * L + lane
        sid1[pl.ds(i * L, L)] = jnp.full((L,), -1, jnp.int32)
        return carry

    lax.fori_loop(0, PER_W // L, pre_body, 0)

    idbase = wid * PER_W

    def cls_body(i, off):
        j = i // 8
        o = (i % 8) * L
        k16 = rowb[j, pl.ds(o, L)] * N + colb[j, pl.ds(o, L)]
        c1 = zbuf[pl.ds(i * L, L)]
        c2 = sval1[pl.ds(i * L, L)]
        d16 = idbase + i * L + lane
        mi = jnp.where((c1 >= 2) & (c2 >= 2), 1, 0)
        total0 = _prefix16(mi, lane)[15]

        def wbody(it, carry):
            off_, mv = carry
            cum = _prefix16(mv, lane)
            sel = jnp.where((mv == 1) & (cum == 1), lane, 0)
            f = _prefix16(sel, lane)[15]
            fs = jnp.full((L,), f, jnp.int32)
            kv = k16.at[fs].get(mode="promise_in_bounds")
            dv = d16.at[fs].get(mode="promise_in_bounds")
            skey1[pl.ds(off_, L)] = kv
            sid1[pl.ds(off_, L)] = dv
            return (off_ + 1, jnp.where(lane == f, 0, mv))

        off2, _ = lax.fori_loop(0, total0, wbody, (off, mi))
        return off2

    ns = lax.fori_loop(0, CH * 8, cls_body, jnp.int32(0))

    def val_body(i, carry):
        v = jnp.where(i * L + lane < ns, 1, 0).astype(jnp.int32)
        sval1[pl.ds(i * L, L)] = v
        return carry

    lax.fori_loop(0, PER_W // L, val_body, 0)

    ng = (ns + L - 1) // L

    def sc_body(g, carry):
        idx16 = skey1[pl.ds(g * L, L)]
        pltpu.async_copy(sid1.at[pl.ds(g * L, L)],
                         map_hbm.at[idx16], sem).wait()
        return carry

    lax.fori_loop(0, ng, sc_body, 0)

    uq = jnp.full((L,), PER_W, jnp.int32) - ns
    urow[pl.ds(0, L)] = jnp.where(lane == 0, uq, 0)
    for t in range(1, 128 // L):
        urow[pl.ds(t * L, L)] = zero16
    pltpu.sync_copy(urow, uniq_hbm.at[wid])
    pltpu.sync_copy(skey1.at[pl.ds(0, PER_W)],
                    skeys_hbm.at[pl.ds(wid * PER_W, PER_W)])
    pltpu.sync_copy(sid1.at[pl.ds(0, PER_W)],
                    sids_hbm.at[pl.ds(wid * PER_W, PER_W)])
    pltpu.sync_copy(sval1, svalid_hbm.at[pl.ds(wid * PER_W, PER_W)])


def _count_body(skeys_hbm, sids_hbm, svalid_hbm, uniq_hbm, map_hbm, cnt_hbm,
                skb, sdb, svb, gotb, urow_b, acc_v, sem):
    cid = lax.axis_index("c")
    sid_ = lax.axis_index("s")
    wid = sid_ * NC + cid
    base = wid * PER_W
    pltpu.sync_copy(skeys_hbm.at[pl.ds(base, PER_W)], skb)
    pltpu.sync_copy(sids_hbm.at[pl.ds(base, PER_W)], sdb)
    pltpu.sync_copy(svalid_hbm.at[pl.ds(base, PER_W)], svb)
    pltpu.sync_copy(uniq_hbm.at[wid], urow_b)
    ns = PER_W - urow_b[pl.ds(0, L)][0]

    def g_body(j, carry):
        pltpu.async_copy(map_hbm.at[skb.at[pl.ds(j * 128, 128)]],
                         gotb.at[pl.ds(j * 128, 128)], sem).wait()
        return carry

    lax.fori_loop(0, (ns + 127) // 128, g_body, 0)

    def body(i, acc):
        o = i * L
        g = gotb[pl.ds(o, L)]
        d = sdb[pl.ds(o, L)]
        v = svb[pl.ds(o, L)]
        one = jnp.ones((L,), jnp.float32)
        zero = jnp.zeros((L,), jnp.float32)
        return acc + jnp.where((g == d) & (v == 1), one, zero)

    acc = lax.fori_loop(0, (ns + L - 1) // L, body,
                        jnp.zeros((L,), jnp.float32))
    acc_v[pl.ds(0, L)] = acc
    zero = jnp.zeros((L,), jnp.float32)
    for t in range(1, 128 // L):
        acc_v[pl.ds(t * L, L)] = zero
    pltpu.sync_copy(acc_v, cnt_hbm.at[wid])


def _colsum_body(x_ref, o_ref):
    s = jnp.sum(x_ref[...], axis=0, keepdims=True)
    o_ref[...] = jnp.broadcast_to(s * (1.0 / K), (K, D))


def _ew_body(cnt_ref, uniq_ref, o_ref):
    total = jnp.sum(cnt_ref[...]) + jnp.sum(uniq_ref[...].astype(jnp.float32))
    w = (total - float(NPAD)) * (1.0 / (K * K))
    o_ref[...] = jnp.full((NW, 128), w, jnp.float32)


def kernel(x, edge_index, edge_attr, W_node, b_node, W_edge, b_edge,
           W_att, b_att):
    row = edge_index[0]
    col = edge_index[1]
    pad_r = jnp.full((NPAD,), N, jnp.int32)
    pad_c = lax.iota(jnp.int32, NPAD)
    rowp = jnp.concatenate([row, pad_r]).reshape(EPAD // 128, 128)
    colp = jnp.concatenate([col, pad_c]).reshape(EPAD // 128, 128)

    classify_k = pl.kernel(
        _classify_body,
        out_type=(
            jax.ShapeDtypeStruct((MAP_SIZE,), jnp.int32),
            jax.ShapeDtypeStruct((EPAD,), jnp.int32),
            jax.ShapeDtypeStruct((EPAD,), jnp.int32),
            jax.ShapeDtypeStruct((EPAD,), jnp.int32),
            jax.ShapeDtypeStruct((NW, 128), jnp.int32),
        ),
        mesh=_mesh,
        scratch_types=[
            pltpu.VMEM((CH, 128), jnp.int32),
            pltpu.VMEM((CH, 128), jnp.int32),
            pltpu.VMEM((CH, 128), jnp.int32),
            pltpu.VMEM((CH, 128), jnp.int32),
            pltpu.VMEM((PER_W + L,), jnp.int32),
            pltpu.VMEM((PER_W + L,), jnp.int32),
            pltpu.VMEM((PER_W,), jnp.int32),
            pltpu.VMEM((ZCH,), jnp.int32),
            pltpu.VMEM((128,), jnp.int32),
            pltpu.VMEM((128,), jnp.int32),
            pltpu.VMEM_SHARED((HSIZE,), jnp.int32),
            pltpu.SemaphoreType.DMA,
        ],
    )
    map_arr, skeys, sids, svalid, uniq = classify_k(rowp, colp)

    count_k = pl.kernel(
        _count_body,
        out_type=jax.ShapeDtypeStruct((NW, 128), jnp.float32),
        mesh=_mesh,
        scratch_types=[pltpu.VMEM((PER_W,), jnp.int32)] * 4
        + [pltpu.VMEM((128,), jnp.int32),
           pltpu.VMEM((128,), jnp.float32), pltpu.SemaphoreType.DMA],
    )
    counts = count_k(skeys, sids, svalid, uniq, map_arr)

    x_pooled = pl.pallas_call(
        _colsum_body,
        out_shape=jax.ShapeDtypeStruct((K, D), jnp.float32),
    )(x)
    ew2d = pl.pallas_call(
        _ew_body,
        out_shape=jax.ShapeDtypeStruct((NW, 128), jnp.float32),
    )(counts, uniq)

    grid = lax.iota(jnp.int32, K * K)
    edge_index_pooled = jnp.stack([grid // K, grid % K], axis=0)
    edge_weight_pooled = ew2d.reshape(K * K)
    batch_pooled = jnp.zeros((K,), jnp.int32)
    return (x_pooled, edge_index_pooled, edge_weight_pooled, batch_pooled)


# speedup vs baseline: 42.2603x; 1.0445x over previous
import jax
import jax.numpy as jnp
from jax import lax
from jax.experimental import pallas as pl
from jax.experimental.pallas import tpu as pltpu
from jax.experimental.pallas import tpu_sc as plsc

N = 10000
E = 160000
D = 128
K = 64

NC = 2
NS = 16
L = 16
NW = NC * NS
EPAD = 163840
PER_W = EPAD // NW
CH = PER_W // 128
SCH = 2 * CH
NPAD = EPAD - E
PADK0 = N * N + NPAD
MAP_SIZE = PADK0 + EPAD
FIRE = 16

ZCH = 8192
HBLK = 160
HSIZE = HBLK * ZCH
HFOLD = (1 << 21) - HSIZE
HMUL1 = -1640531527
HMUL2 = -862048943

_mesh = plsc.VectorSubcoreMesh(core_axis_name="c", subcore_axis_name="s",
                               num_cores=NC, num_subcores=NS)


def _hash16(k, mul, shift):
    h = lax.shift_right_logical(k * mul, shift)
    h = h & ((1 << 21) - 1)
    return jnp.where(h >= HSIZE, h - HFOLD, h)


def _prefix16(v, lane):
    cum = v
    for dsh in (1, 2, 4, 8):
        idx = jnp.maximum(lane - dsh, 0)
        sh = cum.at[idx].get(mode="promise_in_bounds")
        cum = cum + jnp.where(lane >= dsh, sh, 0)
    return cum


def _classify_body(rowp_hbm, colp_hbm, map_hbm, skeys_hbm, sids_hbm,
                   svalid_hbm, uniq_hbm,
                   rowb, colb, hashb, hashb2,
                   skey1, sid1, sval1, zbuf, onesb, urow, hist, sem):
    cid = lax.axis_index("c")
    sid_ = lax.axis_index("s")
    wid = sid_ * NC + cid
    lane = lax.iota(jnp.int32, L)
    zero16 = jnp.zeros((L,), jnp.int32)

    def z_body(i, carry):
        zbuf[pl.ds(i * L, L)] = zero16
        return carry

    lax.fori_loop(0, ZCH // L, z_body, 0)
    zh = []
    for bi in range(HBLK // NS):
        b = sid_ * (HBLK // NS) + bi
        zh.append(pltpu.async_copy(zbuf, hist.at[pl.ds(b * ZCH, ZCH)], sem))

    def ones_body(i, carry):
        onesb[pl.ds(i * L, L)] = jnp.ones((L,), jnp.int32)
        return carry

    lax.fori_loop(0, 128 // L, ones_body, 0)

    def stage_and_hash(first_row):
        pltpu.sync_copy(rowp_hbm.at[pl.ds(first_row, CH)], rowb)
        pltpu.sync_copy(colp_hbm.at[pl.ds(first_row, CH)], colb)

        def keys_body(i, carry):
            j = i // 8
            o = (i % 8) * L
            k = rowb[j, pl.ds(o, L)] * N + colb[j, pl.ds(o, L)]
            hashb[j, pl.ds(o, L)] = _hash16(k, HMUL1, 10)
            hashb2[j, pl.ds(o, L)] = _hash16(k, HMUL2, 9)
            return carry

        lax.fori_loop(0, CH * 8, keys_body, 0)

    stage_and_hash(sid_ * SCH)
    for h in zh:
        h.wait()
    plsc.subcore_barrier()

    for phase in range(2):
        if phase:
            stage_and_hash(sid_ * SCH + phase * CH)
        handles = []
        for hb in (hashb, hashb2):
            for j in range(CH):
                handles.append(
                    pltpu.async_copy(onesb, hist.at[hb.at[j]], sem,
                                     add=True))
                if len(handles) == FIRE:
                    for h in handles:
                        h.wait()
                    handles = []
        for h in handles:
            h.wait()

    plsc.subcore_barrier()

    stage_and_hash(sid_ * SCH + cid * CH)
    handles = []
    for hb, cb in ((hashb, zbuf), (hashb2, sval1)):
        for j in range(CH):
            handles.append(
                pltpu.async_copy(hist.at[hb.at[j]],
                                 cb.at[pl.ds(j * 128, 128)], sem))
            if len(handles) == FIRE:
                for h in handles:
                    h.wait()
                handles = []
    for h in handles:
        h.wait()

    padbase = PADK0 + wid * PER_W

    def pre_body(i, carry):
        skey1[pl.ds(i * L, L)] = padbase + i * L + lane
        sid1[pl.ds(i * L, L)] = jnp.full((L,), -1, jnp.int32)
        return carry

    lax.fori_loop(0, PER_W // L, pre_body, 0)

    idbase = wid * PER_W

    def cls_body(i, off):
        j = i // 8
        o = (i % 8) * L
        k16 = rowb[j, pl.ds(o, L)] * N + colb[j, pl.ds(o, L)]
        c1 = zbuf[pl.ds(i * L, L)]
        c2 = sval1[pl.ds(i * L, L)]
        d16 = idbase + i * L + lane
        mi = jnp.where((c1 >= 2) & (c2 >= 2), 1, 0)
        total0 = _prefix16(mi, lane)[15]

        def wbody(it, carry):
            off_, mv = carry
            cum = _prefix16(mv, lane)
            sel = jnp.where((mv == 1) & (cum == 1), lane, 0)
            f = _prefix16(sel, lane)[15]
            fs = jnp.full((L,), f, jnp.int32)
            kv = k16.at[fs].get(mode="promise_in_bounds")
            dv = d16.at[fs].get(mode="promise_in_bounds")
            skey1[pl.ds(off_, L)] = kv
            sid1[pl.ds(off_, L)] = dv
            return (off_ + 1, jnp.where(lane == f, 0, mv))

        off2, _ = lax.fori_loop(0, total0, wbody, (off, mi))
        return off2

    ns = lax.fori_loop(0, CH * 8, cls_body, jnp.int32(0))

    def val_body(i, carry):
        v = jnp.where(i * L + lane < ns, 1, 0).astype(jnp.int32)
        sval1[pl.ds(i * L, L)] = v
        return carry

    lax.fori_loop(0, PER_W // L, val_body, 0)

    ng = (ns + L - 1) // L

    def sc_body(g, carry):
        idx16 = skey1[pl.ds(g * L, L)]
        pltpu.async_copy(sid1.at[pl.ds(g * L, L)],
                         map_hbm.at[idx16], sem).wait()
        return carry

    lax.fori_loop(0, ng, sc_body, 0)

    uq = jnp.full((L,), PER_W, jnp.int32) - ns
    urow[pl.ds(0, L)] = jnp.where(lane == 0, uq, 0)
    for t in range(1, 128 // L):
        urow[pl.ds(t * L, L)] = zero16
    pltpu.sync_copy(urow, uniq_hbm.at[wid])
    pltpu.sync_copy(skey1.at[pl.ds(0, PER_W)],
                    skeys_hbm.at[pl.ds(wid * PER_W, PER_W)])
    pltpu.sync_copy(sid1.at[pl.ds(0, PER_W)],
                    sids_hbm.at[pl.ds(wid * PER_W, PER_W)])
    pltpu.sync_copy(sval1, svalid_hbm.at[pl.ds(wid * PER_W, PER_W)])


def _count_body(skeys_hbm, sids_hbm, svalid_hbm, uniq_hbm, map_hbm, cnt_hbm,
                skb, sdb, svb, gotb, urow_b, acc_v, sem):
    cid = lax.axis_index("c")
    sid_ = lax.axis_index("s")
    wid = sid_ * NC + cid
    base = wid * PER_W
    pltpu.sync_copy(skeys_hbm.at[pl.ds(base, PER_W)], skb)
    pltpu.sync_copy(sids_hbm.at[pl.ds(base, PER_W)], sdb)
    pltpu.sync_copy(svalid_hbm.at[pl.ds(base, PER_W)], svb)
    pltpu.sync_copy(uniq_hbm.at[wid], urow_b)
    ns = PER_W - urow_b[pl.ds(0, L)][0]

    def g_body(j, carry):
        pltpu.async_copy(map_hbm.at[skb.at[pl.ds(j * 128, 128)]],
                         gotb.at[pl.ds(j * 128, 128)], sem).wait()
        return carry

    lax.fori_loop(0, (ns + 127) // 128, g_body, 0)

    def body(i, acc):
        o = i * L
        g = gotb[pl.ds(o, L)]
        d = sdb[pl.ds(o, L)]
        v = svb[pl.ds(o, L)]
        one = jnp.ones((L,), jnp.float32)
        zero = jnp.zeros((L,), jnp.float32)
        return acc + jnp.where((g == d) & (v == 1), one, zero)

    acc = lax.fori_loop(0, (ns + L - 1) // L, body,
                        jnp.zeros((L,), jnp.float32))
    acc_v[pl.ds(0, L)] = acc
    zero = jnp.zeros((L,), jnp.float32)
    for t in range(1, 128 // L):
        acc_v[pl.ds(t * L, L)] = zero
    pltpu.sync_copy(acc_v, cnt_hbm.at[wid])


def _colsum_body(x_ref, o_ref):
    s = jnp.sum(x_ref[...], axis=0, keepdims=True)
    o_ref[...] = jnp.broadcast_to(s * (1.0 / K), (K, D))


def _ew_body(cnt_ref, uniq_ref, o_ref):
    total = jnp.sum(cnt_ref[...]) + jnp.sum(uniq_ref[...].astype(jnp.float32))
    w = (total - float(NPAD)) * (1.0 / (K * K))
    o_ref[...] = jnp.full((NW, 128), w, jnp.float32)


def kernel(x, edge_index, edge_attr, W_node, b_node, W_edge, b_edge,
           W_att, b_att):
    row = edge_index[0]
    col = edge_index[1]
    pad_r = jnp.full((NPAD,), N, jnp.int32)
    pad_c = lax.iota(jnp.int32, NPAD)
    rowp = jnp.concatenate([row, pad_r]).reshape(EPAD // 128, 128)
    colp = jnp.concatenate([col, pad_c]).reshape(EPAD // 128, 128)

    classify_k = pl.kernel(
        _classify_body,
        out_type=(
            jax.ShapeDtypeStruct((MAP_SIZE,), jnp.int32),
            jax.ShapeDtypeStruct((EPAD,), jnp.int32),
            jax.ShapeDtypeStruct((EPAD,), jnp.int32),
            jax.ShapeDtypeStruct((EPAD,), jnp.int32),
            jax.ShapeDtypeStruct((NW, 128), jnp.int32),
        ),
        mesh=_mesh,
        scratch_types=[
            pltpu.VMEM((CH, 128), jnp.int32),
            pltpu.VMEM((CH, 128), jnp.int32),
            pltpu.VMEM((CH, 128), jnp.int32),
            pltpu.VMEM((CH, 128), jnp.int32),
            pltpu.VMEM((PER_W + L,), jnp.int32),
            pltpu.VMEM((PER_W + L,), jnp.int32),
            pltpu.VMEM((PER_W,), jnp.int32),
            pltpu.VMEM((ZCH,), jnp.int32),
            pltpu.VMEM((128,), jnp.int32),
            pltpu.VMEM((128,), jnp.int32),
            pltpu.VMEM_SHARED((HSIZE,), jnp.int32),
            pltpu.SemaphoreType.DMA,
        ],
    )
    map_arr, skeys, sids, svalid, uniq = classify_k(rowp, colp)

    count_k = pl.kernel(
        _count_body,
        out_type=jax.ShapeDtypeStruct((NW, 128), jnp.float32),
        mesh=_mesh,
        scratch_types=[pltpu.VMEM((PER_W,), jnp.int32)] * 4
        + [pltpu.VMEM((128,), jnp.int32),
           pltpu.VMEM((128,), jnp.float32), pltpu.SemaphoreType.DMA],
    )
    counts = count_k(skeys, sids, svalid, uniq, map_arr)

    x_pooled = pl.pallas_call(
        _colsum_body,
        out_shape=jax.ShapeDtypeStruct((K, D), jnp.float32),
    )(x)
    ew2d = pl.pallas_call(
        _ew_body,
        out_shape=jax.ShapeDtypeStruct((NW, 128), jnp.float32),
    )(counts, uniq)

    grid = lax.iota(jnp.int32, K * K)
    edge_index_pooled = jnp.stack([grid // K, grid % K], axis=0)
    edge_weight_pooled = ew2d.reshape(K * K)
    batch_pooled = jnp.zeros((K,), jnp.int32)
    return (x_pooled, edge_index_pooled, edge_weight_pooled, batch_pooled)


# speedup vs baseline: 43.4924x; 1.0292x over previous
import jax
import jax.numpy as jnp
from jax import lax
from jax.experimental import pallas as pl
from jax.experimental.pallas import tpu as pltpu
from jax.experimental.pallas import tpu_sc as plsc

N = 10000
E = 160000
D = 128
K = 64

NC = 2
NS = 16
L = 16
NW = NC * NS
EPAD = 163840
PER_W = EPAD // NW
CH = PER_W // 128
SCH = 2 * CH
NPAD = EPAD - E
PADK0 = N * N + NPAD
MAP_SIZE = PADK0 + EPAD
FIRE = 16

ZCH = 8192
HBLK = 160
HSIZE = HBLK * ZCH
HFOLD = (1 << 21) - HSIZE
HMUL1 = -1640531527
HMUL2 = -862048943

_mesh = plsc.VectorSubcoreMesh(core_axis_name="c", subcore_axis_name="s",
                               num_cores=NC, num_subcores=NS)


def _hash16(k, mul, shift):
    h = lax.shift_right_logical(k * mul, shift)
    h = h & ((1 << 21) - 1)
    return jnp.where(h >= HSIZE, h - HFOLD, h)


def _prefix16(v, lane):
    cum = v
    for dsh in (1, 2, 4, 8):
        idx = jnp.maximum(lane - dsh, 0)
        sh = cum.at[idx].get(mode="promise_in_bounds")
        cum = cum + jnp.where(lane >= dsh, sh, 0)
    return cum


def _classify_body(rowp_hbm, colp_hbm, map_hbm, skeys_hbm, sids_hbm,
                   svalid_hbm, uniq_hbm,
                   rowb, colb, hashb, hashb2,
                   skey1, sid1, sval1, zbuf, onesb, urow, hist, sem):
    cid = lax.axis_index("c")
    sid_ = lax.axis_index("s")
    wid = sid_ * NC + cid
    lane = lax.iota(jnp.int32, L)
    zero16 = jnp.zeros((L,), jnp.int32)

    def z_body(i, carry):
        zbuf[pl.ds(i * L, L)] = zero16
        return carry

    lax.fori_loop(0, ZCH // L, z_body, 0)
    zh = []
    for bi in range(HBLK // NS):
        b = sid_ * (HBLK // NS) + bi
        zh.append(pltpu.async_copy(zbuf, hist.at[pl.ds(b * ZCH, ZCH)], sem))

    def ones_body(i, carry):
        onesb[pl.ds(i * L, L)] = jnp.ones((L,), jnp.int32)
        return carry

    lax.fori_loop(0, 128 // L, ones_body, 0)

    def stage_and_hash(first_row):
        pltpu.sync_copy(rowp_hbm.at[pl.ds(first_row, CH)], rowb)
        pltpu.sync_copy(colp_hbm.at[pl.ds(first_row, CH)], colb)

        def keys_body(i, carry):
            j = i // 8
            o = (i % 8) * L
            k = rowb[j, pl.ds(o, L)] * N + colb[j, pl.ds(o, L)]
            hashb[j, pl.ds(o, L)] = _hash16(k, HMUL1, 10)
            hashb2[j, pl.ds(o, L)] = _hash16(k, HMUL2, 9)
            return carry

        lax.fori_loop(0, CH * 8, keys_body, 0)

    stage_and_hash(sid_ * SCH + (1 - cid) * CH)
    for h in zh:
        h.wait()
    plsc.subcore_barrier()

    for step in range(2):
        if step:
            stage_and_hash(sid_ * SCH + cid * CH)
        handles = []
        for hb in (hashb, hashb2):
            for j in range(CH):
                handles.append(
                    pltpu.async_copy(onesb, hist.at[hb.at[j]], sem,
                                     add=True))
                if len(handles) == FIRE:
                    for h in handles:
                        h.wait()
                    handles = []
        for h in handles:
            h.wait()

    plsc.subcore_barrier()

    handles = []
    for hb, cb in ((hashb, zbuf), (hashb2, sval1)):
        for j in range(CH):
            handles.append(
                pltpu.async_copy(hist.at[hb.at[j]],
                                 cb.at[pl.ds(j * 128, 128)], sem))
            if len(handles) == FIRE:
                for h in handles:
                    h.wait()
                handles = []
    for h in handles:
        h.wait()

    padbase = PADK0 + wid * PER_W

    def pre_body(i, carry):
        skey1[pl.ds(i * L, L)] = padbase + i * L + lane
        sid1[pl.ds(i * L, L)] = jnp.full((L,), -1, jnp.int32)
        return carry

    lax.fori_loop(0, PER_W // L, pre_body, 0)

    idbase = wid * PER_W

    def cls_body(i, off):
        j = i // 8
        o = (i % 8) * L
        k16 = rowb[j, pl.ds(o, L)] * N + colb[j, pl.ds(o, L)]
        c1 = zbuf[pl.ds(i * L, L)]
        c2 = sval1[pl.ds(i * L, L)]
        d16 = idbase + i * L + lane
        mi = jnp.where((c1 >= 2) & (c2 >= 2), 1, 0)
        total0 = _prefix16(mi, lane)[15]

        def wbody(it, carry):
            off_, mv = carry
            cum = _prefix16(mv, lane)
            sel = jnp.where((mv == 1) & (cum == 1), lane, 0)
            f = _prefix16(sel, lane)[15]
            fs = jnp.full((L,), f, jnp.int32)
            kv = k16.at[fs].get(mode="promise_in_bounds")
            dv = d16.at[fs].get(mode="promise_in_bounds")
            skey1[pl.ds(off_, L)] = kv
            sid1[pl.ds(off_, L)] = dv
            return (off_ + 1, jnp.where(lane == f, 0, mv))

        off2, _ = lax.fori_loop(0, total0, wbody, (off, mi))
        return off2

    ns = lax.fori_loop(0, CH * 8, cls_body, jnp.int32(0))

    def val_body(i, carry):
        v = jnp.where(i * L + lane < ns, 1, 0).astype(jnp.int32)
        sval1[pl.ds(i * L, L)] = v
        return carry

    lax.fori_loop(0, PER_W // L, val_body, 0)

    ng = (ns + L - 1) // L

    def sc_body(g, carry):
        idx16 = skey1[pl.ds(g * L, L)]
        pltpu.async_copy(sid1.at[pl.ds(g * L, L)],
                         map_hbm.at[idx16], sem).wait()
        return carry

    lax.fori_loop(0, ng, sc_body, 0)

    uq = jnp.full((L,), PER_W, jnp.int32) - ns
    urow[pl.ds(0, L)] = jnp.where(lane == 0, uq, 0)
    for t in range(1, 128 // L):
        urow[pl.ds(t * L, L)] = zero16
    pltpu.sync_copy(urow, uniq_hbm.at[wid])
    pltpu.sync_copy(skey1.at[pl.ds(0, PER_W)],
                    skeys_hbm.at[pl.ds(wid * PER_W, PER_W)])
    pltpu.sync_copy(sid1.at[pl.ds(0, PER_W)],
                    sids_hbm.at[pl.ds(wid * PER_W, PER_W)])
    pltpu.sync_copy(sval1, svalid_hbm.at[pl.ds(wid * PER_W, PER_W)])


def _count_body(skeys_hbm, sids_hbm, svalid_hbm, uniq_hbm, map_hbm, cnt_hbm,
                skb, sdb, svb, gotb, urow_b, acc_v, sem):
    cid = lax.axis_index("c")
    sid_ = lax.axis_index("s")
    wid = sid_ * NC + cid
    base = wid * PER_W
    pltpu.sync_copy(skeys_hbm.at[pl.ds(base, PER_W)], skb)
    pltpu.sync_copy(sids_hbm.at[pl.ds(base, PER_W)], sdb)
    pltpu.sync_copy(svalid_hbm.at[pl.ds(base, PER_W)], svb)
    pltpu.sync_copy(uniq_hbm.at[wid], urow_b)
    ns = PER_W - urow_b[pl.ds(0, L)][0]

    def g_body(j, carry):
        pltpu.async_copy(map_hbm.at[skb.at[pl.ds(j * 128, 128)]],
                         gotb.at[pl.ds(j * 128, 128)], sem).wait()
        return carry

    lax.fori_loop(0, (ns + 127) // 128, g_body, 0)

    def body(i, acc):
        o = i * L
        g = gotb[pl.ds(o, L)]
        d = sdb[pl.ds(o, L)]
        v = svb[pl.ds(o, L)]
        one = jnp.ones((L,), jnp.float32)
        zero = jnp.zeros((L,), jnp.float32)
        return acc + jnp.where((g == d) & (v == 1), one, zero)

    acc = lax.fori_loop(0, (ns + L - 1) // L, body,
                        jnp.zeros((L,), jnp.float32))
    acc_v[pl.ds(0, L)] = acc
    zero = jnp.zeros((L,), jnp.float32)
    for t in range(1, 128 // L):
        acc_v[pl.ds(t * L, L)] = zero
    pltpu.sync_copy(acc_v, cnt_hbm.at[wid])


def _colsum_body(x_ref, o_ref):
    s = jnp.sum(x_ref[...], axis=0, keepdims=True)
    o_ref[...] = jnp.broadcast_to(s * (1.0 / K), (K, D))


def _ew_body(cnt_ref, uniq_ref, o_ref):
    total = jnp.sum(cnt_ref[...]) + jnp.sum(uniq_ref[...].astype(jnp.float32))
    w = (total - float(NPAD)) * (1.0 / (K * K))
    o_ref[...] = jnp.full((NW, 128), w, jnp.float32)


def kernel(x, edge_index, edge_attr, W_node, b_node, W_edge, b_edge,
           W_att, b_att):
    row = edge_index[0]
    col = edge_index[1]
    pad_r = jnp.full((NPAD,), N, jnp.int32)
    pad_c = lax.iota(jnp.int32, NPAD)
    rowp = jnp.concatenate([row, pad_r]).reshape(EPAD // 128, 128)
    colp = jnp.concatenate([col, pad_c]).reshape(EPAD // 128, 128)

    classify_k = pl.kernel(
        _classify_body,
        out_type=(
            jax.ShapeDtypeStruct((MAP_SIZE,), jnp.int32),
            jax.ShapeDtypeStruct((EPAD,), jnp.int32),
            jax.ShapeDtypeStruct((EPAD,), jnp.int32),
            jax.ShapeDtypeStruct((EPAD,), jnp.int32),
            jax.ShapeDtypeStruct((NW, 128), jnp.int32),
        ),
        mesh=_mesh,
        scratch_types=[
            pltpu.VMEM((CH, 128), jnp.int32),
            pltpu.VMEM((CH, 128), jnp.int32),
            pltpu.VMEM((CH, 128), jnp.int32),
            pltpu.VMEM((CH, 128), jnp.int32),
            pltpu.VMEM((PER_W + L,), jnp.int32),
            pltpu.VMEM((PER_W + L,), jnp.int32),
            pltpu.VMEM((PER_W,), jnp.int32),
            pltpu.VMEM((ZCH,), jnp.int32),
            pltpu.VMEM((128,), jnp.int32),
            pltpu.VMEM((128,), jnp.int32),
            pltpu.VMEM_SHARED((HSIZE,), jnp.int32),
            pltpu.SemaphoreType.DMA,
        ],
    )
    map_arr, skeys, sids, svalid, uniq = classify_k(rowp, colp)

    count_k = pl.kernel(
        _count_body,
        out_type=jax.ShapeDtypeStruct((NW, 128), jnp.float32),
        mesh=_mesh,
        scratch_types=[pltpu.VMEM((PER_W,), jnp.int32)] * 4
        + [pltpu.VMEM((128,), jnp.int32),
           pltpu.VMEM((128,), jnp.float32), pltpu.SemaphoreType.DMA],
    )
    counts = count_k(skeys, sids, svalid, uniq, map_arr)

    x_pooled = pl.pallas_call(
        _colsum_body,
        out_shape=jax.ShapeDtypeStruct((K, D), jnp.float32),
    )(x)
    ew2d = pl.pallas_call(
        _ew_body,
        out_shape=jax.ShapeDtypeStruct((NW, 128), jnp.float32),
    )(counts, uniq)

    grid = lax.iota(jnp.int32, K * K)
    edge_index_pooled = jnp.stack([grid // K, grid % K], axis=0)
    edge_weight_pooled = ew2d.reshape(K * K)
    batch_pooled = jnp.zeros((K,), jnp.int32)
    return (x_pooled, edge_index_pooled, edge_weight_pooled, batch_pooled)


# speedup vs baseline: 43.8326x; 1.0078x over previous
import jax
import jax.numpy as jnp
from jax import lax
from jax.experimental import pallas as pl
from jax.experimental.pallas import tpu as pltpu
from jax.experimental.pallas import tpu_sc as plsc

N = 10000
E = 160000
D = 128
K = 64

NC = 2
NS = 16
L = 16
NW = NC * NS
EPAD = 163840
PER_W = EPAD // NW
CH = PER_W // 128
SCH = 2 * CH
NPAD = EPAD - E
PADK0 = N * N + NPAD
MAP_SIZE = PADK0 + EPAD
FIRE = 32

ZCH = 8192
HBLK = 160
HSIZE = HBLK * ZCH
HFOLD = (1 << 21) - HSIZE
HMUL1 = -1640531527
HMUL2 = -862048943

_mesh = plsc.VectorSubcoreMesh(core_axis_name="c", subcore_axis_name="s",
                               num_cores=NC, num_subcores=NS)


def _hash16(k, mul, shift):
    h = lax.shift_right_logical(k * mul, shift)
    h = h & ((1 << 21) - 1)
    return jnp.where(h >= HSIZE, h - HFOLD, h)


def _prefix16(v, lane):
    cum = v
    for dsh in (1, 2, 4, 8):
        idx = jnp.maximum(lane - dsh, 0)
        sh = cum.at[idx].get(mode="promise_in_bounds")
        cum = cum + jnp.where(lane >= dsh, sh, 0)
    return cum


def _classify_body(rowp_hbm, colp_hbm, map_hbm, skeys_hbm, sids_hbm,
                   svalid_hbm, uniq_hbm,
                   rowb, colb, hashb, hashb2,
                   skey1, sid1, sval1, zbuf, onesb, urow, hist, sem):
    cid = lax.axis_index("c")
    sid_ = lax.axis_index("s")
    wid = sid_ * NC + cid
    lane = lax.iota(jnp.int32, L)
    zero16 = jnp.zeros((L,), jnp.int32)

    def z_body(i, carry):
        zbuf[pl.ds(i * L, L)] = zero16
        return carry

    lax.fori_loop(0, ZCH // L, z_body, 0)
    zh = []
    for bi in range(HBLK // NS):
        b = sid_ * (HBLK // NS) + bi
        zh.append(pltpu.async_copy(zbuf, hist.at[pl.ds(b * ZCH, ZCH)], sem))

    def ones_body(i, carry):
        onesb[pl.ds(i * L, L)] = jnp.ones((L,), jnp.int32)
        return carry

    lax.fori_loop(0, 128 // L, ones_body, 0)

    def stage_and_hash(first_row):
        pltpu.sync_copy(rowp_hbm.at[pl.ds(first_row, CH)], rowb)
        pltpu.sync_copy(colp_hbm.at[pl.ds(first_row, CH)], colb)

        def keys_body(i, carry):
            j = i // 8
            o = (i % 8) * L
            k = rowb[j, pl.ds(o, L)] * N + colb[j, pl.ds(o, L)]
            hashb[j, pl.ds(o, L)] = _hash16(k, HMUL1, 10)
            hashb2[j, pl.ds(o, L)] = _hash16(k, HMUL2, 9)
            return carry

        lax.fori_loop(0, CH * 8, keys_body, 0)

    stage_and_hash(sid_ * SCH + (1 - cid) * CH)
    for h in zh:
        h.wait()
    plsc.subcore_barrier()

    for step in range(2):
        if step:
            stage_and_hash(sid_ * SCH + cid * CH)
        handles = []
        for hb in (hashb, hashb2):
            for j in range(CH):
                handles.append(
                    pltpu.async_copy(onesb, hist.at[hb.at[j]], sem,
                                     add=True))
                if len(handles) == FIRE:
                    for h in handles:
                        h.wait()
                    handles = []
        for h in handles:
            h.wait()

    plsc.subcore_barrier()

    handles = []
    for hb, cb in ((hashb, zbuf), (hashb2, sval1)):
        for j in range(CH):
            handles.append(
                pltpu.async_copy(hist.at[hb.at[j]],
                                 cb.at[pl.ds(j * 128, 128)], sem))
            if len(handles) == FIRE:
                for h in handles:
                    h.wait()
                handles = []
    for h in handles:
        h.wait()

    padbase = PADK0 + wid * PER_W

    def pre_body(i, carry):
        skey1[pl.ds(i * L, L)] = padbase + i * L + lane
        sid1[pl.ds(i * L, L)] = jnp.full((L,), -1, jnp.int32)
        return carry

    lax.fori_loop(0, PER_W // L, pre_body, 0)

    idbase = wid * PER_W

    def cls_body(i, off):
        j = i // 8
        o = (i % 8) * L
        k16 = rowb[j, pl.ds(o, L)] * N + colb[j, pl.ds(o, L)]
        c1 = zbuf[pl.ds(i * L, L)]
        c2 = sval1[pl.ds(i * L, L)]
        d16 = idbase + i * L + lane
        mi = jnp.where((c1 >= 2) & (c2 >= 2), 1, 0)
        total0 = _prefix16(mi, lane)[15]

        def wbody(it, carry):
            off_, mv = carry
            cum = _prefix16(mv, lane)
            sel = jnp.where((mv == 1) & (cum == 1), lane, 0)
            f = _prefix16(sel, lane)[15]
            fs = jnp.full((L,), f, jnp.int32)
            kv = k16.at[fs].get(mode="promise_in_bounds")
            dv = d16.at[fs].get(mode="promise_in_bounds")
            skey1[pl.ds(off_, L)] = kv
            sid1[pl.ds(off_, L)] = dv
            return (off_ + 1, jnp.where(lane == f, 0, mv))

        off2, _ = lax.fori_loop(0, total0, wbody, (off, mi))
        return off2

    ns = lax.fori_loop(0, CH * 8, cls_body, jnp.int32(0))

    def val_body(i, carry):
        v = jnp.where(i * L + lane < ns, 1, 0).astype(jnp.int32)
        sval1[pl.ds(i * L, L)] = v
        return carry

    lax.fori_loop(0, PER_W // L, val_body, 0)

    ng = (ns + L - 1) // L

    def sc_body(g, carry):
        idx16 = skey1[pl.ds(g * L, L)]
        pltpu.async_copy(sid1.at[pl.ds(g * L, L)],
                         map_hbm.at[idx16], sem).wait()
        return carry

    lax.fori_loop(0, ng, sc_body, 0)

    uq = jnp.full((L,), PER_W, jnp.int32) - ns
    urow[pl.ds(0, L)] = jnp.where(lane == 0, uq, 0)
    for t in range(1, 128 // L):
        urow[pl.ds(t * L, L)] = zero16
    pltpu.sync_copy(urow, uniq_hbm.at[wid])
    pltpu.sync_copy(skey1.at[pl.ds(0, PER_W)],
                    skeys_hbm.at[pl.ds(wid * PER_W, PER_W)])
    pltpu.sync_copy(sid1.at[pl.ds(0, PER_W)],
                    sids_hbm.at[pl.ds(wid * PER_W, PER_W)])
    pltpu.sync_copy(sval1, svalid_hbm.at[pl.ds(wid * PER_W, PER_W)])


def _count_body(skeys_hbm, sids_hbm, svalid_hbm, uniq_hbm, map_hbm, cnt_hbm,
                skb, sdb, svb, gotb, urow_b, acc_v, sem):
    cid = lax.axis_index("c")
    sid_ = lax.axis_index("s")
    wid = sid_ * NC + cid
    base = wid * PER_W
    pltpu.sync_copy(skeys_hbm.at[pl.ds(base, PER_W)], skb)
    pltpu.sync_copy(sids_hbm.at[pl.ds(base, PER_W)], sdb)
    pltpu.sync_copy(svalid_hbm.at[pl.ds(base, PER_W)], svb)
    pltpu.sync_copy(uniq_hbm.at[wid], urow_b)
    ns = PER_W - urow_b[pl.ds(0, L)][0]

    def g_body(j, carry):
        pltpu.async_copy(map_hbm.at[skb.at[pl.ds(j * 128, 128)]],
                         gotb.at[pl.ds(j * 128, 128)], sem).wait()
        return carry

    lax.fori_loop(0, (ns + 127) // 128, g_body, 0)

    def body(i, acc):
        o = i * L
        g = gotb[pl.ds(o, L)]
        d = sdb[pl.ds(o, L)]
        v = svb[pl.ds(o, L)]
        one = jnp.ones((L,), jnp.float32)
        zero = jnp.zeros((L,), jnp.float32)
        return acc + jnp.where((g == d) & (v == 1), one, zero)

    acc = lax.fori_loop(0, (ns + L - 1) // L, body,
                        jnp.zeros((L,), jnp.float32))
    acc_v[pl.ds(0, L)] = acc
    zero = jnp.zeros((L,), jnp.float32)
    for t in range(1, 128 // L):
        acc_v[pl.ds(t * L, L)] = zero
    pltpu.sync_copy(acc_v, cnt_hbm.at[wid])


def _colsum_body(x_ref, o_ref):
    s = jnp.sum(x_ref[...], axis=0, keepdims=True)
    o_ref[...] = jnp.broadcast_to(s * (1.0 / K), (K, D))


def _ew_body(cnt_ref, uniq_ref, o_ref):
    total = jnp.sum(cnt_ref[...]) + jnp.sum(uniq_ref[...].astype(jnp.float32))
    w = (total - float(NPAD)) * (1.0 / (K * K))
    o_ref[...] = jnp.full((NW, 128), w, jnp.float32)


def kernel(x, edge_index, edge_attr, W_node, b_node, W_edge, b_edge,
           W_att, b_att):
    row = edge_index[0]
    col = edge_index[1]
    pad_r = jnp.full((NPAD,), N, jnp.int32)
    pad_c = lax.iota(jnp.int32, NPAD)
    rowp = jnp.concatenate([row, pad_r]).reshape(EPAD // 128, 128)
    colp = jnp.concatenate([col, pad_c]).reshape(EPAD // 128, 128)

    classify_k = pl.kernel(
        _classify_body,
        out_type=(
            jax.ShapeDtypeStruct((MAP_SIZE,), jnp.int32),
            jax.ShapeDtypeStruct((EPAD,), jnp.int32),
            jax.ShapeDtypeStruct((EPAD,), jnp.int32),
            jax.ShapeDtypeStruct((EPAD,), jnp.int32),
            jax.ShapeDtypeStruct((NW, 128), jnp.int32),
        ),
        mesh=_mesh,
        scratch_types=[
            pltpu.VMEM((CH, 128), jnp.int32),
            pltpu.VMEM((CH, 128), jnp.int32),
            pltpu.VMEM((CH, 128), jnp.int32),
            pltpu.VMEM((CH, 128), jnp.int32),
            pltpu.VMEM((PER_W + L,), jnp.int32),
            pltpu.VMEM((PER_W + L,), jnp.int32),
            pltpu.VMEM((PER_W,), jnp.int32),
            pltpu.VMEM((ZCH,), jnp.int32),
            pltpu.VMEM((128,), jnp.int32),
            pltpu.VMEM((128,), jnp.int32),
            pltpu.VMEM_SHARED((HSIZE,), jnp.int32),
            pltpu.SemaphoreType.DMA,
        ],
    )
    map_arr, skeys, sids, svalid, uniq = classify_k(rowp, colp)

    count_k = pl.kernel(
        _count_body,
        out_type=jax.ShapeDtypeStruct((NW, 128), jnp.float32),
        mesh=_mesh,
        scratch_types=[pltpu.VMEM((PER_W,), jnp.int32)] * 4
        + [pltpu.VMEM((128,), jnp.int32),
           pltpu.VMEM((128,), jnp.float32), pltpu.SemaphoreType.DMA],
    )
    counts = count_k(skeys, sids, svalid, uniq, map_arr)

    x_pooled = pl.pallas_call(
        _colsum_body,
        out_shape=jax.ShapeDtypeStruct((K, D), jnp.float32),
    )(x)
    ew2d = pl.pallas_call(
        _ew_body,
        out_shape=jax.ShapeDtypeStruct((NW, 128), jnp.float32),
    )(counts, uniq)

    grid = lax.iota(jnp.int32, K * K)
    edge_index_pooled = jnp.stack([grid // K, grid % K], axis=0)
    edge_weight_pooled = ew2d.reshape(K * K)
    batch_pooled = jnp.zeros((K,), jnp.int32)
    return (x_pooled, edge_index_pooled, edge_weight_pooled, batch_pooled)
